# Initial kernel scaffold; baseline (speedup 1.0000x reference)
import jax
import jax.numpy as jnp
from jax.experimental import pallas as pl


def kernel(x, Wr1, br1, Wr2, br2, W1, b1, W2, b2):
    raise NotImplementedError("write your pallas kernel here")


# speedup vs baseline: 1.3588x; 1.3588x over previous
import functools

import jax
import jax.numpy as jnp
from jax import lax
from jax.experimental import pallas as pl
from jax.experimental.pallas import tpu as pltpu
from jax.experimental.pallas import tpu_sc as plsc

EMBED = 1024
HIDDEN = 4096
NEXP = 8
S = 2048
EP = 128
BLK_M = 128
NBLK = 40
NSLOT = NBLK * BLK_M
NEG = -3.0e38

@functools.lru_cache(maxsize=None)
def _vector_mesh():
    return plsc.VectorSubcoreMesh(core_axis_name="c", subcore_axis_name="s")


def _router_body(x_ref, wr1_ref, br1_ref, wr2_ref, br2_ref, st_ref):
    h = jnp.maximum(
        jnp.dot(x_ref[...], wr1_ref[...], preferred_element_type=jnp.float32)
        + br1_ref[...], 0.0)
    st = lax.dot_general(wr2_ref[...], h, (((0,), (1,)), ((), ())),
                         preferred_element_type=jnp.float32)
    st_ref[...] = st + br2_ref[...]


def _router(x2, Wr1, br1r, Wr2p, br2c):
    mb = 512
    return pl.pallas_call(
        _router_body,
        grid=(S // mb,),
        in_specs=[
            pl.BlockSpec((mb, EMBED), lambda m: (m, 0)),
            pl.BlockSpec((EMBED, HIDDEN), lambda m: (0, 0)),
            pl.BlockSpec((1, HIDDEN), lambda m: (0, 0)),
            pl.BlockSpec((HIDDEN, EP), lambda m: (0, 0)),
            pl.BlockSpec((EP, 1), lambda m: (0, 0)),
        ],
        out_specs=pl.BlockSpec((EP, mb), lambda m: (0, m)),
        out_shape=jax.ShapeDtypeStruct((EP, S), jnp.float32),
    )(x2, Wr1, br1r, Wr2p, br2c)


def _routing_body(st_ref, pos_ref, meta_ref, gs_ref,
                  i1_ref, i2_ref, g1_ref, g2_ref, r1_ref, r2_ref):
    TB = 128
    NB = S // TB
    row = lax.broadcasted_iota(jnp.int32, (EP, TB), 0)
    ut = (lax.broadcasted_iota(jnp.int32, (TB, TB), 0)
          < lax.broadcasted_iota(jnp.int32, (TB, TB), 1)).astype(jnp.float32)

    def pass1(b, carry):
        p1, p2 = carry
        sl = pl.ds(b * TB, TB)
        sb = st_ref[:, sl]
        m1 = jnp.max(sb, axis=0, keepdims=True)
        i1 = jnp.min(jnp.where(sb == m1, row, EP), axis=0, keepdims=True)
        s2 = jnp.where(row == i1, NEG, sb)
        m2 = jnp.max(s2, axis=0, keepdims=True)
        i2 = jnp.min(jnp.where(s2 == m2, row, EP), axis=0, keepdims=True)
        e = jnp.exp(m2 - m1)
        g1 = 1.0 / (1.0 + e)
        o1 = (row == i1).astype(jnp.float32)
        o2 = (row == i2).astype(jnp.float32)
        c1 = lax.dot_general(o1, ut, (((1,), (0,)), ((), ())),
                             preferred_element_type=jnp.float32) + p1
        c2 = lax.dot_general(o2, ut, (((1,), (0,)), ((), ())),
                             preferred_element_type=jnp.float32) + p2
        i1_ref[:, sl] = i1
        i2_ref[:, sl] = i2
        g1_ref[:, sl] = g1
        g2_ref[:, sl] = e * g1
        r1_ref[:, sl] = jnp.sum(o1 * c1, axis=0, keepdims=True)
        r2_ref[:, sl] = jnp.sum(o2 * c2, axis=0, keepdims=True)
        return (p1 + jnp.sum(o1, axis=1, keepdims=True),
                p2 + jnp.sum(o2, axis=1, keepdims=True))

    zero = jnp.zeros((EP, 1), jnp.float32)
    cnt1, cnt2 = lax.fori_loop(0, NB, pass1, (zero, zero))

    cnt = cnt1 + cnt2
    blocks = jnp.floor((cnt + (BLK_M - 1)) * (1.0 / BLK_M))
    lt = (lax.broadcasted_iota(jnp.int32, (EP, EP), 1)
          < lax.broadcasted_iota(jnp.int32, (EP, EP), 0)).astype(jnp.float32)
    bexc = lax.dot_general(lt, blocks, (((1,), (0,)), ((), ())),
                           preferred_element_type=jnp.float32)
    off = bexc * float(BLK_M)
    bend = (bexc + blocks).astype(jnp.int32)
    total = jnp.sum(blocks).astype(jnp.int32)

    mrow = lax.broadcasted_iota(jnp.int32, (1, NBLK), 1)
    mcl = jnp.minimum(mrow, total - 1)
    be = jnp.sum((bend <= mcl).astype(jnp.int32), axis=0, keepdims=True)
    meta_ref[0:1, :] = be
    meta_ref[1:2, :] = (mrow < total).astype(jnp.int32)

    def pass2(b, carry):
        sl = pl.ds(b * TB, TB)
        o1 = (row == i1_ref[:, sl]).astype(jnp.float32)
        o2 = (row == i2_ref[:, sl]).astype(jnp.float32)
        pos1 = r1_ref[:, sl] + jnp.sum(o1 * off, axis=0, keepdims=True)
        pos2 = r2_ref[:, sl] + jnp.sum(o2 * (off + cnt1), axis=0, keepdims=True)
        pos_ref[0:1, sl] = pos1.astype(jnp.int32)
        pos_ref[1:2, sl] = pos2.astype(jnp.int32)
        r1_ref[:, sl] = pos1
        r2_ref[:, sl] = pos2
        return carry

    lax.fori_loop(0, NB, pass2, 0)

    p1row = r1_ref[...]
    p2row = r2_ref[...]
    g1row = g1_ref[...]
    g2row = g2_ref[...]
    pcol = lax.broadcasted_iota(jnp.int32, (BLK_M, 1), 0).astype(jnp.float32)
    for m in range(NBLK):
        pc = pcol + float(m * BLK_M)
        eq1 = (pc == p1row).astype(jnp.float32)
        eq2 = (pc == p2row).astype(jnp.float32)
        gs = (lax.dot_general(eq1, g1row, (((1,), (1,)), ((), ())),
                              preferred_element_type=jnp.float32)
              + lax.dot_general(eq2, g2row, (((1,), (1,)), ((), ())),
                                preferred_element_type=jnp.float32))
        gs_ref[m] = gs


def _routing(scoresT):
    return pl.pallas_call(
        _routing_body,
        in_specs=[pl.BlockSpec((EP, S), lambda: (0, 0))],
        out_specs=[
            pl.BlockSpec((2, S), lambda: (0, 0)),
            pl.BlockSpec((2, NBLK), lambda: (0, 0)),
            pl.BlockSpec((NBLK, BLK_M, 1), lambda: (0, 0, 0)),
        ],
        out_shape=[
            jax.ShapeDtypeStruct((2, S), jnp.int32),
            jax.ShapeDtypeStruct((2, NBLK), jnp.int32),
            jax.ShapeDtypeStruct((NBLK, BLK_M, 1), jnp.float32),
        ],
        scratch_shapes=[pltpu.VMEM((1, S), jnp.int32),
                        pltpu.VMEM((1, S), jnp.int32),
                        pltpu.VMEM((1, S), jnp.float32),
                        pltpu.VMEM((1, S), jnp.float32),
                        pltpu.VMEM((1, S), jnp.float32),
                        pltpu.VMEM((1, S), jnp.float32)],
    )(scoresT)


@functools.lru_cache(maxsize=None)
def _make_dispatch():
    @functools.partial(
        pl.kernel,
        out_type=jax.ShapeDtypeStruct((NSLOT, EMBED), jnp.float32),
        mesh=_vector_mesh(),
        scratch_types=[pltpu.VMEM((2, 64), jnp.int32),
                       pltpu.VMEM((64, EMBED), jnp.float32),
                       pltpu.SemaphoreType.DMA],
    )
    def _dispatch(x_hbm, pos_hbm, xg_hbm, idx_v, rows_v, sem):
        w = lax.axis_index("s") * 2 + lax.axis_index("c")
        k = w // 16
        i = w % 16
        pltpu.sync_copy(pos_hbm.at[k, i], idx_v)

        @pl.loop(0, 2)
        def _(sb):
            t0 = i * 128 + sb * 64
            pltpu.sync_copy(x_hbm.at[pl.ds(t0, 64)], rows_v)
            pltpu.async_copy(rows_v, xg_hbm.at[idx_v.at[sb]], sem).wait()

    return _dispatch


def _phase1_body(meta_ref, xg_ref, w1_ref, b1_ref, h_ref):
    @pl.when(meta_ref[1, pl.program_id(0)] == 1)
    def _():
        h_ref[...] = jnp.maximum(
            jnp.dot(xg_ref[...], w1_ref[0],
                    preferred_element_type=jnp.float32) + b1_ref[0], 0.0)


def _phase1(meta, Xg, W1, b1r):
    return pl.pallas_call(
        _phase1_body,
        grid_spec=pltpu.PrefetchScalarGridSpec(
            num_scalar_prefetch=1,
            grid=(NBLK,),
            in_specs=[
                pl.BlockSpec((BLK_M, EMBED), lambda m, meta: (m, 0)),
                pl.BlockSpec((1, EMBED, HIDDEN), lambda m, meta: (meta[0, m], 0, 0)),
                pl.BlockSpec((1, 1, HIDDEN), lambda m, meta: (meta[0, m], 0, 0)),
            ],
            out_specs=pl.BlockSpec((BLK_M, HIDDEN), lambda m, meta: (m, 0)),
        ),
        out_shape=jax.ShapeDtypeStruct((NSLOT, HIDDEN), jnp.float32),
    )(meta, Xg, W1, b1r.reshape(NEXP, 1, HIDDEN))


def _phase2_body(meta_ref, h_ref, w2_ref, b2_ref, gs_ref, y_ref):
    @pl.when(meta_ref[1, pl.program_id(0)] == 1)
    def _():
        y = jnp.dot(h_ref[...], w2_ref[0],
                    preferred_element_type=jnp.float32) + b2_ref[0]
        y_ref[...] = y * gs_ref[0]


def _phase2(meta, H, W2, b2r, GS):
    return pl.pallas_call(
        _phase2_body,
        grid_spec=pltpu.PrefetchScalarGridSpec(
            num_scalar_prefetch=1,
            grid=(NBLK,),
            in_specs=[
                pl.BlockSpec((BLK_M, HIDDEN), lambda m, meta: (m, 0)),
                pl.BlockSpec((1, HIDDEN, EMBED), lambda m, meta: (meta[0, m], 0, 0)),
                pl.BlockSpec((1, 1, EMBED), lambda m, meta: (meta[0, m], 0, 0)),
                pl.BlockSpec((1, BLK_M, 1), lambda m, meta: (m, 0, 0)),
            ],
            out_specs=pl.BlockSpec((BLK_M, EMBED), lambda m, meta: (m, 0)),
        ),
        out_shape=jax.ShapeDtypeStruct((NSLOT, EMBED), jnp.float32),
    )(meta, H, W2, b2r.reshape(NEXP, 1, EMBED), GS)


@functools.lru_cache(maxsize=None)
def _make_combine():
    @functools.partial(
        pl.kernel,
        out_type=jax.ShapeDtypeStruct((S, EMBED), jnp.float32),
        mesh=_vector_mesh(),
        scratch_types=[pltpu.VMEM((2, 32), jnp.int32),
                       pltpu.VMEM((2, 32), jnp.int32),
                       pltpu.VMEM((32, EMBED), jnp.float32),
                       pltpu.VMEM((32, EMBED), jnp.float32),
                       pltpu.VMEM((32, EMBED), jnp.float32),
                       pltpu.SemaphoreType.DMA,
                       pltpu.SemaphoreType.DMA],
    )
    def _combine(yg_hbm, pos_hbm, out_hbm, p0, p1, r0, r1, ov, sem0, sem1):
        w = lax.axis_index("s") * 2 + lax.axis_index("c")
        pltpu.sync_copy(pos_hbm.at[0, w], p0)
        pltpu.sync_copy(pos_hbm.at[1, w], p1)

        @pl.loop(0, 2)
        def _(sb):
            cp0 = pltpu.async_copy(yg_hbm.at[p0.at[sb]], r0, sem0)
            cp1 = pltpu.async_copy(yg_hbm.at[p1.at[sb]], r1, sem1)
            cp0.wait()
            cp1.wait()

            @pl.loop(0, 32)
            def _(j):
                @pl.loop(0, EMBED // 16)
                def _(cc):
                    sl = pl.ds(cc * 16, 16)
                    ov[j, sl] = r0[j, sl] + r1[j, sl]

            pltpu.sync_copy(ov, out_hbm.at[pl.ds(w * 64 + sb * 32, 32)])

    return _combine


def kernel(x, Wr1, br1, Wr2, br2, W1, b1, W2, b2):
    B = x.shape[0]
    x2 = x.reshape(S, EMBED)
    Wr2p = jnp.zeros((HIDDEN, EP), jnp.float32).at[:, :NEXP].set(Wr2)
    br2c = jnp.full((EP, 1), NEG / 4, jnp.float32).at[:NEXP, 0].set(br2)

    scoresT = _router(x2, Wr1, br1.reshape(1, HIDDEN), Wr2p, br2c)
    pos, meta, GS = _routing(scoresT)
    Xg = _make_dispatch()(x2, pos.reshape(2, 16, 2, 64))
    H = _phase1(meta, Xg, W1, b1)
    Yg = _phase2(meta, H, W2, b2, GS)
    out = _make_combine()(Yg, pos.reshape(2, 32, 2, 32))
    return out.reshape(B, S, EMBED)


# speedup vs baseline: 1.3595x; 1.0005x over previous
import functools

import jax
import jax.numpy as jnp
from jax import lax
from jax.experimental import pallas as pl
from jax.experimental.pallas import tpu as pltpu
from jax.experimental.pallas import tpu_sc as plsc

EMBED = 1024
HIDDEN = 4096
NEXP = 8
S = 2048
EP = 128
BLK_M = 128
NBLK = 40
NSLOT = NBLK * BLK_M
NEG = -3.0e38

@functools.lru_cache(maxsize=None)
def _vector_mesh():
    return plsc.VectorSubcoreMesh(core_axis_name="c", subcore_axis_name="s")


def _router_body(x_ref, wr1_ref, br1_ref, wr2_ref, br2_ref, st_ref):
    h = jnp.maximum(
        jnp.dot(x_ref[...], wr1_ref[...], preferred_element_type=jnp.float32)
        + br1_ref[...], 0.0)
    st = lax.dot_general(wr2_ref[...], h, (((0,), (1,)), ((), ())),
                         preferred_element_type=jnp.float32)
    st_ref[...] = st + br2_ref[...]


def _router(x2, Wr1, br1r, Wr2p, br2c):
    mb = 512
    return pl.pallas_call(
        _router_body,
        grid=(S // mb,),
        in_specs=[
            pl.BlockSpec((mb, EMBED), lambda m: (m, 0)),
            pl.BlockSpec((EMBED, HIDDEN), lambda m: (0, 0)),
            pl.BlockSpec((1, HIDDEN), lambda m: (0, 0)),
            pl.BlockSpec((HIDDEN, EP), lambda m: (0, 0)),
            pl.BlockSpec((EP, 1), lambda m: (0, 0)),
        ],
        out_specs=pl.BlockSpec((EP, mb), lambda m: (0, m)),
        out_shape=jax.ShapeDtypeStruct((EP, S), jnp.float32),
        compiler_params=pltpu.CompilerParams(
            dimension_semantics=("parallel",)),
    )(x2, Wr1, br1r, Wr2p, br2c)


def _routing_body(st_ref, pos_ref, meta_ref, gs_ref,
                  i1_ref, i2_ref, g1_ref, g2_ref, r1_ref, r2_ref):
    TB = 128
    NB = S // TB
    row = lax.broadcasted_iota(jnp.int32, (EP, TB), 0)
    ut = (lax.broadcasted_iota(jnp.int32, (TB, TB), 0)
          < lax.broadcasted_iota(jnp.int32, (TB, TB), 1)).astype(jnp.float32)

    def pass1(b, carry):
        p1, p2 = carry
        sl = pl.ds(b * TB, TB)
        sb = st_ref[:, sl]
        m1 = jnp.max(sb, axis=0, keepdims=True)
        i1 = jnp.min(jnp.where(sb == m1, row, EP), axis=0, keepdims=True)
        s2 = jnp.where(row == i1, NEG, sb)
        m2 = jnp.max(s2, axis=0, keepdims=True)
        i2 = jnp.min(jnp.where(s2 == m2, row, EP), axis=0, keepdims=True)
        e = jnp.exp(m2 - m1)
        g1 = 1.0 / (1.0 + e)
        o1 = (row == i1).astype(jnp.float32)
        o2 = (row == i2).astype(jnp.float32)
        c1 = lax.dot_general(o1, ut, (((1,), (0,)), ((), ())),
                             preferred_element_type=jnp.float32) + p1
        c2 = lax.dot_general(o2, ut, (((1,), (0,)), ((), ())),
                             preferred_element_type=jnp.float32) + p2
        i1_ref[:, sl] = i1
        i2_ref[:, sl] = i2
        g1_ref[:, sl] = g1
        g2_ref[:, sl] = e * g1
        r1_ref[:, sl] = jnp.sum(o1 * c1, axis=0, keepdims=True)
        r2_ref[:, sl] = jnp.sum(o2 * c2, axis=0, keepdims=True)
        return (p1 + jnp.sum(o1, axis=1, keepdims=True),
                p2 + jnp.sum(o2, axis=1, keepdims=True))

    zero = jnp.zeros((EP, 1), jnp.float32)
    cnt1, cnt2 = lax.fori_loop(0, NB, pass1, (zero, zero))

    cnt = cnt1 + cnt2
    blocks = jnp.floor((cnt + (BLK_M - 1)) * (1.0 / BLK_M))
    lt = (lax.broadcasted_iota(jnp.int32, (EP, EP), 1)
          < lax.broadcasted_iota(jnp.int32, (EP, EP), 0)).astype(jnp.float32)
    bexc = lax.dot_general(lt, blocks, (((1,), (0,)), ((), ())),
                           preferred_element_type=jnp.float32)
    off = bexc * float(BLK_M)
    bend = (bexc + blocks).astype(jnp.int32)
    total = jnp.sum(blocks).astype(jnp.int32)

    mrow = lax.broadcasted_iota(jnp.int32, (1, NBLK), 1)
    mcl = jnp.minimum(mrow, total - 1)
    be = jnp.sum((bend <= mcl).astype(jnp.int32), axis=0, keepdims=True)
    meta_ref[0:1, :] = be
    meta_ref[1:2, :] = (mrow < total).astype(jnp.int32)

    def pass2(b, carry):
        sl = pl.ds(b * TB, TB)
        o1 = (row == i1_ref[:, sl]).astype(jnp.float32)
        o2 = (row == i2_ref[:, sl]).astype(jnp.float32)
        pos1 = r1_ref[:, sl] + jnp.sum(o1 * off, axis=0, keepdims=True)
        pos2 = r2_ref[:, sl] + jnp.sum(o2 * (off + cnt1), axis=0, keepdims=True)
        pos_ref[0:1, sl] = pos1.astype(jnp.int32)
        pos_ref[1:2, sl] = pos2.astype(jnp.int32)
        r1_ref[:, sl] = pos1
        r2_ref[:, sl] = pos2
        return carry

    lax.fori_loop(0, NB, pass2, 0)

    p1row = r1_ref[...]
    p2row = r2_ref[...]
    g1row = g1_ref[...]
    g2row = g2_ref[...]
    pcol = lax.broadcasted_iota(jnp.int32, (BLK_M, 1), 0).astype(jnp.float32)
    for m in range(NBLK):
        pc = pcol + float(m * BLK_M)
        eq1 = (pc == p1row).astype(jnp.float32)
        eq2 = (pc == p2row).astype(jnp.float32)
        gs = (lax.dot_general(eq1, g1row, (((1,), (1,)), ((), ())),
                              preferred_element_type=jnp.float32)
              + lax.dot_general(eq2, g2row, (((1,), (1,)), ((), ())),
                                preferred_element_type=jnp.float32))
        gs_ref[m] = gs


def _routing(scoresT):
    return pl.pallas_call(
        _routing_body,
        in_specs=[pl.BlockSpec((EP, S), lambda: (0, 0))],
        out_specs=[
            pl.BlockSpec((2, S), lambda: (0, 0)),
            pl.BlockSpec((2, NBLK), lambda: (0, 0)),
            pl.BlockSpec((NBLK, BLK_M, 1), lambda: (0, 0, 0)),
        ],
        out_shape=[
            jax.ShapeDtypeStruct((2, S), jnp.int32),
            jax.ShapeDtypeStruct((2, NBLK), jnp.int32),
            jax.ShapeDtypeStruct((NBLK, BLK_M, 1), jnp.float32),
        ],
        scratch_shapes=[pltpu.VMEM((1, S), jnp.int32),
                        pltpu.VMEM((1, S), jnp.int32),
                        pltpu.VMEM((1, S), jnp.float32),
                        pltpu.VMEM((1, S), jnp.float32),
                        pltpu.VMEM((1, S), jnp.float32),
                        pltpu.VMEM((1, S), jnp.float32)],
    )(scoresT)


@functools.lru_cache(maxsize=None)
def _make_dispatch():
    @functools.partial(
        pl.kernel,
        out_type=jax.ShapeDtypeStruct((NSLOT, EMBED), jnp.float32),
        mesh=_vector_mesh(),
        scratch_types=[pltpu.VMEM((2, 64), jnp.int32),
                       pltpu.VMEM((64, EMBED), jnp.float32),
                       pltpu.SemaphoreType.DMA],
    )
    def _dispatch(x_hbm, pos_hbm, xg_hbm, idx_v, rows_v, sem):
        w = lax.axis_index("s") * 2 + lax.axis_index("c")
        k = w // 16
        i = w % 16
        pltpu.sync_copy(pos_hbm.at[k, i], idx_v)

        @pl.loop(0, 2)
        def _(sb):
            t0 = i * 128 + sb * 64
            pltpu.sync_copy(x_hbm.at[pl.ds(t0, 64)], rows_v)
            pltpu.async_copy(rows_v, xg_hbm.at[idx_v.at[sb]], sem).wait()

    return _dispatch


def _phase1_body(meta_ref, xg_ref, w1_ref, b1_ref, h_ref):
    @pl.when(meta_ref[1, pl.program_id(0)] == 1)
    def _():
        h_ref[...] = jnp.maximum(
            jnp.dot(xg_ref[...], w1_ref[0],
                    preferred_element_type=jnp.float32) + b1_ref[0], 0.0)


def _phase1(meta, Xg, W1, b1r):
    return pl.pallas_call(
        _phase1_body,
        grid_spec=pltpu.PrefetchScalarGridSpec(
            num_scalar_prefetch=1,
            grid=(NBLK,),
            in_specs=[
                pl.BlockSpec((BLK_M, EMBED), lambda m, meta: (m, 0)),
                pl.BlockSpec((1, EMBED, HIDDEN), lambda m, meta: (meta[0, m], 0, 0)),
                pl.BlockSpec((1, 1, HIDDEN), lambda m, meta: (meta[0, m], 0, 0)),
            ],
            out_specs=pl.BlockSpec((BLK_M, HIDDEN), lambda m, meta: (m, 0)),
        ),
        out_shape=jax.ShapeDtypeStruct((NSLOT, HIDDEN), jnp.float32),
        compiler_params=pltpu.CompilerParams(
            dimension_semantics=("parallel",)),
    )(meta, Xg, W1, b1r.reshape(NEXP, 1, HIDDEN))


def _phase2_body(meta_ref, h_ref, w2_ref, b2_ref, gs_ref, y_ref):
    @pl.when(meta_ref[1, pl.program_id(0)] == 1)
    def _():
        y = jnp.dot(h_ref[...], w2_ref[0],
                    preferred_element_type=jnp.float32) + b2_ref[0]
        y_ref[...] = y * gs_ref[0]


def _phase2(meta, H, W2, b2r, GS):
    return pl.pallas_call(
        _phase2_body,
        grid_spec=pltpu.PrefetchScalarGridSpec(
            num_scalar_prefetch=1,
            grid=(NBLK,),
            in_specs=[
                pl.BlockSpec((BLK_M, HIDDEN), lambda m, meta: (m, 0)),
                pl.BlockSpec((1, HIDDEN, EMBED), lambda m, meta: (meta[0, m], 0, 0)),
                pl.BlockSpec((1, 1, EMBED), lambda m, meta: (meta[0, m], 0, 0)),
                pl.BlockSpec((1, BLK_M, 1), lambda m, meta: (m, 0, 0)),
            ],
            out_specs=pl.BlockSpec((BLK_M, EMBED), lambda m, meta: (m, 0)),
        ),
        out_shape=jax.ShapeDtypeStruct((NSLOT, EMBED), jnp.float32),
        compiler_params=pltpu.CompilerParams(
            dimension_semantics=("parallel",)),
    )(meta, H, W2, b2r.reshape(NEXP, 1, EMBED), GS)


@functools.lru_cache(maxsize=None)
def _make_combine():
    @functools.partial(
        pl.kernel,
        out_type=jax.ShapeDtypeStruct((S, EMBED), jnp.float32),
        mesh=_vector_mesh(),
        scratch_types=[pltpu.VMEM((2, 32), jnp.int32),
                       pltpu.VMEM((2, 32), jnp.int32),
                       pltpu.VMEM((32, EMBED), jnp.float32),
                       pltpu.VMEM((32, EMBED), jnp.float32),
                       pltpu.VMEM((32, EMBED), jnp.float32),
                       pltpu.SemaphoreType.DMA,
                       pltpu.SemaphoreType.DMA],
    )
    def _combine(yg_hbm, pos_hbm, out_hbm, p0, p1, r0, r1, ov, sem0, sem1):
        w = lax.axis_index("s") * 2 + lax.axis_index("c")
        pltpu.sync_copy(pos_hbm.at[0, w], p0)
        pltpu.sync_copy(pos_hbm.at[1, w], p1)

        @pl.loop(0, 2)
        def _(sb):
            cp0 = pltpu.async_copy(yg_hbm.at[p0.at[sb]], r0, sem0)
            cp1 = pltpu.async_copy(yg_hbm.at[p1.at[sb]], r1, sem1)
            cp0.wait()
            cp1.wait()

            @pl.loop(0, 32)
            def _(j):
                @pl.loop(0, EMBED // 16)
                def _(cc):
                    sl = pl.ds(cc * 16, 16)
                    ov[j, sl] = r0[j, sl] + r1[j, sl]

            pltpu.sync_copy(ov, out_hbm.at[pl.ds(w * 64 + sb * 32, 32)])

    return _combine


def kernel(x, Wr1, br1, Wr2, br2, W1, b1, W2, b2):
    B = x.shape[0]
    x2 = x.reshape(S, EMBED)
    Wr2p = jnp.zeros((HIDDEN, EP), jnp.float32).at[:, :NEXP].set(Wr2)
    br2c = jnp.full((EP, 1), NEG / 4, jnp.float32).at[:NEXP, 0].set(br2)

    scoresT = _router(x2, Wr1, br1.reshape(1, HIDDEN), Wr2p, br2c)
    pos, meta, GS = _routing(scoresT)
    Xg = _make_dispatch()(x2, pos.reshape(2, 16, 2, 64))
    H = _phase1(meta, Xg, W1, b1)
    Yg = _phase2(meta, H, W2, b2, GS)
    out = _make_combine()(Yg, pos.reshape(2, 32, 2, 32))
    return out.reshape(B, S, EMBED)


# speedup vs baseline: 1.4124x; 1.0390x over previous
import functools

import jax
import jax.numpy as jnp
from jax import lax
from jax.experimental import pallas as pl
from jax.experimental.pallas import tpu as pltpu
from jax.experimental.pallas import tpu_sc as plsc

EMBED = 1024
HIDDEN = 4096
NEXP = 8
S = 2048
EP = 128
BLK_M = 128
NBLK = 40
NSLOT = NBLK * BLK_M
NEG = -3.0e38

@functools.lru_cache(maxsize=None)
def _vector_mesh():
    return plsc.VectorSubcoreMesh(core_axis_name="c", subcore_axis_name="s")


def _router_body(x_ref, wr1_ref, br1_ref, wr2_ref, br2_ref, st_ref):
    h = jnp.maximum(
        jnp.dot(x_ref[...], wr1_ref[...], preferred_element_type=jnp.float32)
        + br1_ref[...], 0.0)
    st = lax.dot_general(wr2_ref[...], h, (((0,), (1,)), ((), ())),
                         preferred_element_type=jnp.float32)
    st_ref[...] = st + br2_ref[...]


def _router(x2, Wr1, br1r, Wr2p, br2c):
    mb = 512
    return pl.pallas_call(
        _router_body,
        grid=(S // mb,),
        in_specs=[
            pl.BlockSpec((mb, EMBED), lambda m: (m, 0)),
            pl.BlockSpec((EMBED, HIDDEN), lambda m: (0, 0)),
            pl.BlockSpec((1, HIDDEN), lambda m: (0, 0)),
            pl.BlockSpec((HIDDEN, EP), lambda m: (0, 0)),
            pl.BlockSpec((EP, 1), lambda m: (0, 0)),
        ],
        out_specs=pl.BlockSpec((EP, mb), lambda m: (0, m)),
        out_shape=jax.ShapeDtypeStruct((EP, S), jnp.float32),
        compiler_params=pltpu.CompilerParams(
            dimension_semantics=("parallel",)),
    )(x2, Wr1, br1r, Wr2p, br2c)


def _routing_body(st_ref, pos_ref, meta_ref, gs_ref,
                  i1_ref, i2_ref, g1_ref, g2_ref, r1_ref, r2_ref):
    TB = 128
    NB = S // TB
    row = lax.broadcasted_iota(jnp.int32, (EP, TB), 0)
    ut = (lax.broadcasted_iota(jnp.int32, (TB, TB), 0)
          < lax.broadcasted_iota(jnp.int32, (TB, TB), 1)).astype(jnp.float32)

    def pass1(b, carry):
        p1, p2 = carry
        sl = pl.ds(b * TB, TB)
        sb = st_ref[:, sl]
        m1 = jnp.max(sb, axis=0, keepdims=True)
        i1 = jnp.min(jnp.where(sb == m1, row, EP), axis=0, keepdims=True)
        s2 = jnp.where(row == i1, NEG, sb)
        m2 = jnp.max(s2, axis=0, keepdims=True)
        i2 = jnp.min(jnp.where(s2 == m2, row, EP), axis=0, keepdims=True)
        e = jnp.exp(m2 - m1)
        g1 = 1.0 / (1.0 + e)
        o1 = (row == i1).astype(jnp.float32)
        o2 = (row == i2).astype(jnp.float32)
        c1 = lax.dot_general(o1, ut, (((1,), (0,)), ((), ())),
                             preferred_element_type=jnp.float32) + p1
        c2 = lax.dot_general(o2, ut, (((1,), (0,)), ((), ())),
                             preferred_element_type=jnp.float32) + p2
        i1_ref[:, sl] = i1
        i2_ref[:, sl] = i2
        g1_ref[:, sl] = g1
        g2_ref[:, sl] = e * g1
        r1_ref[:, sl] = jnp.sum(o1 * c1, axis=0, keepdims=True)
        r2_ref[:, sl] = jnp.sum(o2 * c2, axis=0, keepdims=True)
        return (p1 + jnp.sum(o1, axis=1, keepdims=True),
                p2 + jnp.sum(o2, axis=1, keepdims=True))

    zero = jnp.zeros((EP, 1), jnp.float32)
    cnt1, cnt2 = lax.fori_loop(0, NB, pass1, (zero, zero))

    cnt = cnt1 + cnt2
    blocks = jnp.floor((cnt + (BLK_M - 1)) * (1.0 / BLK_M))
    lt = (lax.broadcasted_iota(jnp.int32, (EP, EP), 1)
          < lax.broadcasted_iota(jnp.int32, (EP, EP), 0)).astype(jnp.float32)
    bexc = lax.dot_general(lt, blocks, (((1,), (0,)), ((), ())),
                           preferred_element_type=jnp.float32)
    off = bexc * float(BLK_M)
    bend = (bexc + blocks).astype(jnp.int32)
    total = jnp.sum(blocks).astype(jnp.int32)

    mrow = lax.broadcasted_iota(jnp.int32, (1, NBLK), 1)
    mcl = jnp.minimum(mrow, total - 1)
    be = jnp.sum((bend <= mcl).astype(jnp.int32), axis=0, keepdims=True)
    meta_ref[0:1, :] = be
    meta_ref[1:2, :] = (mrow < total).astype(jnp.int32)

    def pass2(b, carry):
        sl = pl.ds(b * TB, TB)
        o1 = (row == i1_ref[:, sl]).astype(jnp.float32)
        o2 = (row == i2_ref[:, sl]).astype(jnp.float32)
        pos1 = r1_ref[:, sl] + jnp.sum(o1 * off, axis=0, keepdims=True)
        pos2 = r2_ref[:, sl] + jnp.sum(o2 * (off + cnt1), axis=0, keepdims=True)
        pos_ref[0:1, sl] = pos1.astype(jnp.int32)
        pos_ref[1:2, sl] = pos2.astype(jnp.int32)
        r1_ref[:, sl] = pos1
        r2_ref[:, sl] = pos2
        return carry

    lax.fori_loop(0, NB, pass2, 0)

    p1row = r1_ref[...]
    p2row = r2_ref[...]
    g1row = g1_ref[...]
    g2row = g2_ref[...]
    pcol = lax.broadcasted_iota(jnp.int32, (BLK_M, 1), 0).astype(jnp.float32)
    for m in range(NBLK):
        pc = pcol + float(m * BLK_M)
        eq1 = (pc == p1row).astype(jnp.float32)
        eq2 = (pc == p2row).astype(jnp.float32)
        gs = (lax.dot_general(eq1, g1row, (((1,), (1,)), ((), ())),
                              preferred_element_type=jnp.float32)
              + lax.dot_general(eq2, g2row, (((1,), (1,)), ((), ())),
                                preferred_element_type=jnp.float32))
        gs_ref[m] = gs


def _routing(scoresT):
    return pl.pallas_call(
        _routing_body,
        in_specs=[pl.BlockSpec((EP, S), lambda: (0, 0))],
        out_specs=[
            pl.BlockSpec((2, S), lambda: (0, 0)),
            pl.BlockSpec((2, NBLK), lambda: (0, 0)),
            pl.BlockSpec((NBLK, BLK_M, 1), lambda: (0, 0, 0)),
        ],
        out_shape=[
            jax.ShapeDtypeStruct((2, S), jnp.int32),
            jax.ShapeDtypeStruct((2, NBLK), jnp.int32),
            jax.ShapeDtypeStruct((NBLK, BLK_M, 1), jnp.float32),
        ],
        scratch_shapes=[pltpu.VMEM((1, S), jnp.int32),
                        pltpu.VMEM((1, S), jnp.int32),
                        pltpu.VMEM((1, S), jnp.float32),
                        pltpu.VMEM((1, S), jnp.float32),
                        pltpu.VMEM((1, S), jnp.float32),
                        pltpu.VMEM((1, S), jnp.float32)],
    )(scoresT)


@functools.lru_cache(maxsize=None)
def _make_dispatch():
    @functools.partial(
        pl.kernel,
        out_type=jax.ShapeDtypeStruct((NSLOT, EMBED), jnp.float32),
        mesh=_vector_mesh(),
        scratch_types=[pltpu.VMEM((2, 64), jnp.int32),
                       pltpu.VMEM((64, EMBED), jnp.float32),
                       pltpu.SemaphoreType.DMA],
    )
    def _dispatch(x_hbm, pos_hbm, xg_hbm, idx_v, rows_v, sem):
        w = lax.axis_index("s") * 2 + lax.axis_index("c")
        k = w // 16
        i = w % 16
        pltpu.sync_copy(pos_hbm.at[k, i], idx_v)

        @pl.loop(0, 2)
        def _(sb):
            t0 = i * 128 + sb * 64
            pltpu.sync_copy(x_hbm.at[pl.ds(t0, 64)], rows_v)
            pltpu.async_copy(rows_v, xg_hbm.at[idx_v.at[sb]], sem).wait()

    return _dispatch


def _phase1_body(meta_ref, xg_ref, w1_ref, b1_ref, h_ref):
    @pl.when(meta_ref[1, pl.program_id(0)] == 1)
    def _():
        h = jnp.dot(xg_ref[...], w1_ref[0],
                    preferred_element_type=jnp.float32) + b1_ref[0]
        h_ref[...] = jnp.maximum(h, 0.0).astype(jnp.bfloat16)


def _phase1(meta, Xg, W1, b1r):
    return pl.pallas_call(
        _phase1_body,
        grid_spec=pltpu.PrefetchScalarGridSpec(
            num_scalar_prefetch=1,
            grid=(NBLK,),
            in_specs=[
                pl.BlockSpec((BLK_M, EMBED), lambda m, meta: (m, 0)),
                pl.BlockSpec((1, EMBED, HIDDEN), lambda m, meta: (meta[0, m], 0, 0)),
                pl.BlockSpec((1, 1, HIDDEN), lambda m, meta: (meta[0, m], 0, 0)),
            ],
            out_specs=pl.BlockSpec((BLK_M, HIDDEN), lambda m, meta: (m, 0)),
        ),
        out_shape=jax.ShapeDtypeStruct((NSLOT, HIDDEN), jnp.bfloat16),
        compiler_params=pltpu.CompilerParams(
            dimension_semantics=("parallel",)),
    )(meta, Xg, W1, b1r.reshape(NEXP, 1, HIDDEN))


def _phase2_body(meta_ref, h_ref, w2_ref, b2_ref, gs_ref, y_ref):
    @pl.when(meta_ref[1, pl.program_id(0)] == 1)
    def _():
        y = jnp.dot(h_ref[...], w2_ref[0],
                    preferred_element_type=jnp.float32) + b2_ref[0]
        y_ref[...] = y * gs_ref[0]


def _phase2(meta, H, W2, b2r, GS):
    return pl.pallas_call(
        _phase2_body,
        grid_spec=pltpu.PrefetchScalarGridSpec(
            num_scalar_prefetch=1,
            grid=(NBLK,),
            in_specs=[
                pl.BlockSpec((BLK_M, HIDDEN), lambda m, meta: (m, 0)),
                pl.BlockSpec((1, HIDDEN, EMBED), lambda m, meta: (meta[0, m], 0, 0)),
                pl.BlockSpec((1, 1, EMBED), lambda m, meta: (meta[0, m], 0, 0)),
                pl.BlockSpec((1, BLK_M, 1), lambda m, meta: (m, 0, 0)),
            ],
            out_specs=pl.BlockSpec((BLK_M, EMBED), lambda m, meta: (m, 0)),
        ),
        out_shape=jax.ShapeDtypeStruct((NSLOT, EMBED), jnp.float32),
        compiler_params=pltpu.CompilerParams(
            dimension_semantics=("parallel",)),
    )(meta, H, W2, b2r.reshape(NEXP, 1, EMBED), GS)


@functools.lru_cache(maxsize=None)
def _make_combine():
    @functools.partial(
        pl.kernel,
        out_type=jax.ShapeDtypeStruct((S, EMBED), jnp.float32),
        mesh=_vector_mesh(),
        scratch_types=[pltpu.VMEM((2, 32), jnp.int32),
                       pltpu.VMEM((2, 32), jnp.int32),
                       pltpu.VMEM((32, EMBED), jnp.float32),
                       pltpu.VMEM((32, EMBED), jnp.float32),
                       pltpu.VMEM((32, EMBED), jnp.float32),
                       pltpu.SemaphoreType.DMA,
                       pltpu.SemaphoreType.DMA],
    )
    def _combine(yg_hbm, pos_hbm, out_hbm, p0, p1, r0, r1, ov, sem0, sem1):
        w = lax.axis_index("s") * 2 + lax.axis_index("c")
        pltpu.sync_copy(pos_hbm.at[0, w], p0)
        pltpu.sync_copy(pos_hbm.at[1, w], p1)

        @pl.loop(0, 2)
        def _(sb):
            cp0 = pltpu.async_copy(yg_hbm.at[p0.at[sb]], r0, sem0)
            cp1 = pltpu.async_copy(yg_hbm.at[p1.at[sb]], r1, sem1)
            cp0.wait()
            cp1.wait()

            @pl.loop(0, 32)
            def _(j):
                @pl.loop(0, EMBED // 16)
                def _(cc):
                    sl = pl.ds(cc * 16, 16)
                    ov[j, sl] = r0[j, sl] + r1[j, sl]

            pltpu.sync_copy(ov, out_hbm.at[pl.ds(w * 64 + sb * 32, 32)])

    return _combine


def kernel(x, Wr1, br1, Wr2, br2, W1, b1, W2, b2):
    B = x.shape[0]
    x2 = x.reshape(S, EMBED)
    Wr2p = jnp.zeros((HIDDEN, EP), jnp.float32).at[:, :NEXP].set(Wr2)
    br2c = jnp.full((EP, 1), NEG / 4, jnp.float32).at[:NEXP, 0].set(br2)

    scoresT = _router(x2, Wr1, br1.reshape(1, HIDDEN), Wr2p, br2c)
    pos, meta, GS = _routing(scoresT)
    Xg = _make_dispatch()(x2, pos.reshape(2, 16, 2, 64))
    H = _phase1(meta, Xg, W1, b1)
    Yg = _phase2(meta, H, W2, b2, GS)
    out = _make_combine()(Yg, pos.reshape(2, 32, 2, 32))
    return out.reshape(B, S, EMBED)


# speedup vs baseline: 1.4390x; 1.0188x over previous
import functools

import jax
import jax.numpy as jnp
from jax import lax
from jax.experimental import pallas as pl
from jax.experimental.pallas import tpu as pltpu
from jax.experimental.pallas import tpu_sc as plsc

EMBED = 1024
HIDDEN = 4096
NEXP = 8
S = 2048
EP = 128
BLK_M = 128
NBLK = 40
NSLOT = NBLK * BLK_M
NEG = -3.0e38

@functools.lru_cache(maxsize=None)
def _vector_mesh():
    return plsc.VectorSubcoreMesh(core_axis_name="c", subcore_axis_name="s")


def _router_body(x_ref, wr1_ref, br1_ref, wr2_ref, br2_ref, st_ref):
    h = jnp.maximum(
        jnp.dot(x_ref[...], wr1_ref[...], preferred_element_type=jnp.float32)
        + br1_ref[...], 0.0)
    st = lax.dot_general(wr2_ref[...], h, (((0,), (1,)), ((), ())),
                         preferred_element_type=jnp.float32)
    st_ref[...] = st + br2_ref[...]


def _router(x2, Wr1, br1r, Wr2p, br2c):
    mb = 512
    return pl.pallas_call(
        _router_body,
        grid=(S // mb,),
        in_specs=[
            pl.BlockSpec((mb, EMBED), lambda m: (m, 0)),
            pl.BlockSpec((EMBED, HIDDEN), lambda m: (0, 0)),
            pl.BlockSpec((1, HIDDEN), lambda m: (0, 0)),
            pl.BlockSpec((HIDDEN, EP), lambda m: (0, 0)),
            pl.BlockSpec((EP, 1), lambda m: (0, 0)),
        ],
        out_specs=pl.BlockSpec((EP, mb), lambda m: (0, m)),
        out_shape=jax.ShapeDtypeStruct((EP, S), jnp.float32),
        compiler_params=pltpu.CompilerParams(
            dimension_semantics=("parallel",)),
    )(x2, Wr1, br1r, Wr2p, br2c)


def _routing_body(st_ref, pos_ref, meta_ref, gs_ref,
                  i1_ref, i2_ref, g1_ref, g2_ref, r1_ref, r2_ref):
    TB = 128
    NB = S // TB
    row = lax.broadcasted_iota(jnp.int32, (EP, TB), 0)
    ut = (lax.broadcasted_iota(jnp.int32, (TB, TB), 0)
          < lax.broadcasted_iota(jnp.int32, (TB, TB), 1)).astype(jnp.float32)

    def pass1(b, carry):
        p1, p2 = carry
        sl = pl.ds(b * TB, TB)
        sb = st_ref[:, sl]
        m1 = jnp.max(sb, axis=0, keepdims=True)
        i1 = jnp.min(jnp.where(sb == m1, row, EP), axis=0, keepdims=True)
        s2 = jnp.where(row == i1, NEG, sb)
        m2 = jnp.max(s2, axis=0, keepdims=True)
        i2 = jnp.min(jnp.where(s2 == m2, row, EP), axis=0, keepdims=True)
        e = jnp.exp(m2 - m1)
        g1 = 1.0 / (1.0 + e)
        o1 = (row == i1).astype(jnp.float32)
        o2 = (row == i2).astype(jnp.float32)
        c1 = lax.dot_general(o1, ut, (((1,), (0,)), ((), ())),
                             preferred_element_type=jnp.float32) + p1
        c2 = lax.dot_general(o2, ut, (((1,), (0,)), ((), ())),
                             preferred_element_type=jnp.float32) + p2
        i1_ref[:, sl] = i1
        i2_ref[:, sl] = i2
        g1_ref[:, sl] = g1
        g2_ref[:, sl] = e * g1
        r1_ref[:, sl] = jnp.sum(o1 * c1, axis=0, keepdims=True)
        r2_ref[:, sl] = jnp.sum(o2 * c2, axis=0, keepdims=True)
        return (p1 + jnp.sum(o1, axis=1, keepdims=True),
                p2 + jnp.sum(o2, axis=1, keepdims=True))

    zero = jnp.zeros((EP, 1), jnp.float32)
    cnt1, cnt2 = lax.fori_loop(0, NB, pass1, (zero, zero))

    cnt = cnt1 + cnt2
    blocks = jnp.floor((cnt + (BLK_M - 1)) * (1.0 / BLK_M))
    lt = (lax.broadcasted_iota(jnp.int32, (EP, EP), 1)
          < lax.broadcasted_iota(jnp.int32, (EP, EP), 0)).astype(jnp.float32)
    bexc = lax.dot_general(lt, blocks, (((1,), (0,)), ((), ())),
                           preferred_element_type=jnp.float32)
    off = bexc * float(BLK_M)
    bend = (bexc + blocks).astype(jnp.int32)
    total = jnp.sum(blocks).astype(jnp.int32)

    mrow = lax.broadcasted_iota(jnp.int32, (1, NBLK), 1)
    mcl = jnp.minimum(mrow, total - 1)
    be = jnp.sum((bend <= mcl).astype(jnp.int32), axis=0, keepdims=True)
    uf = (blocks > 0.0).astype(jnp.float32)
    uexc = lax.dot_general(lt, uf, (((1,), (0,)), ((), ())),
                           preferred_element_type=jnp.float32)
    mrowf = mrow.astype(jnp.float32)
    mclf = mcl.astype(jnp.float32)
    rf = jnp.sum(uf * (bexc <= mclf).astype(jnp.float32), axis=0, keepdims=True)
    first = jnp.sum(uf * (bexc == mrowf).astype(jnp.float32), axis=0,
                    keepdims=True)
    rowidx = lax.broadcasted_iota(jnp.int32, (EP, 1), 0).astype(jnp.float32)
    eqn = uf * (uexc == rf).astype(jnp.float32)
    vsum = jnp.sum(rowidx * eqn, axis=0, keepdims=True)
    exist = jnp.sum(eqn, axis=0, keepdims=True)
    nf = jnp.where(exist > 0.0, vsum, -1.0)
    meta_ref[0:1, :] = be
    meta_ref[1:2, :] = (rf.astype(jnp.int32) - 1) % 2
    meta_ref[2:3, :] = first.astype(jnp.int32)
    meta_ref[3:4, :] = nf.astype(jnp.int32)
    meta_ref[4:5, :] = jnp.broadcast_to(total, (1, NBLK))

    def pass2(b, carry):
        sl = pl.ds(b * TB, TB)
        o1 = (row == i1_ref[:, sl]).astype(jnp.float32)
        o2 = (row == i2_ref[:, sl]).astype(jnp.float32)
        pos1 = r1_ref[:, sl] + jnp.sum(o1 * off, axis=0, keepdims=True)
        pos2 = r2_ref[:, sl] + jnp.sum(o2 * (off + cnt1), axis=0, keepdims=True)
        pos_ref[0:1, sl] = pos1.astype(jnp.int32)
        pos_ref[1:2, sl] = pos2.astype(jnp.int32)
        r1_ref[:, sl] = pos1
        r2_ref[:, sl] = pos2
        return carry

    lax.fori_loop(0, NB, pass2, 0)

    p1row = r1_ref[...]
    p2row = r2_ref[...]
    g1row = g1_ref[...]
    g2row = g2_ref[...]
    pcol = lax.broadcasted_iota(jnp.int32, (BLK_M, 1), 0).astype(jnp.float32)
    for m in range(NBLK):
        pc = pcol + float(m * BLK_M)
        eq1 = (pc == p1row).astype(jnp.float32)
        eq2 = (pc == p2row).astype(jnp.float32)
        gs = (lax.dot_general(eq1, g1row, (((1,), (1,)), ((), ())),
                              preferred_element_type=jnp.float32)
              + lax.dot_general(eq2, g2row, (((1,), (1,)), ((), ())),
                                preferred_element_type=jnp.float32))
        gs_ref[m] = gs


def _routing(scoresT):
    return pl.pallas_call(
        _routing_body,
        in_specs=[pl.BlockSpec((EP, S), lambda: (0, 0))],
        out_specs=[
            pl.BlockSpec((2, S), lambda: (0, 0)),
            pl.BlockSpec((5, NBLK), lambda: (0, 0)),
            pl.BlockSpec((NBLK, BLK_M, 1), lambda: (0, 0, 0)),
        ],
        out_shape=[
            jax.ShapeDtypeStruct((2, S), jnp.int32),
            jax.ShapeDtypeStruct((5, NBLK), jnp.int32),
            jax.ShapeDtypeStruct((NBLK, BLK_M, 1), jnp.float32),
        ],
        scratch_shapes=[pltpu.VMEM((1, S), jnp.int32),
                        pltpu.VMEM((1, S), jnp.int32),
                        pltpu.VMEM((1, S), jnp.float32),
                        pltpu.VMEM((1, S), jnp.float32),
                        pltpu.VMEM((1, S), jnp.float32),
                        pltpu.VMEM((1, S), jnp.float32)],
    )(scoresT)


@functools.lru_cache(maxsize=None)
def _make_dispatch():
    @functools.partial(
        pl.kernel,
        out_type=jax.ShapeDtypeStruct((NSLOT, EMBED), jnp.float32),
        mesh=_vector_mesh(),
        scratch_types=[pltpu.VMEM((2, 64), jnp.int32),
                       pltpu.VMEM((64, EMBED), jnp.float32),
                       pltpu.SemaphoreType.DMA],
    )
    def _dispatch(x_hbm, pos_hbm, xg_hbm, idx_v, rows_v, sem):
        w = lax.axis_index("s") * 2 + lax.axis_index("c")
        k = w // 16
        i = w % 16
        pltpu.sync_copy(pos_hbm.at[k, i], idx_v)

        @pl.loop(0, 2)
        def _(sb):
            t0 = i * 128 + sb * 64
            pltpu.sync_copy(x_hbm.at[pl.ds(t0, 64)], rows_v)
            pltpu.async_copy(rows_v, xg_hbm.at[idx_v.at[sb]], sem).wait()

    return _dispatch


def _make_grouped_mm(din, dout, out_dtype, phase):
    def body(meta_ref, src_ref, w_ref, b_ref, *rest):
        if phase == 2:
            gs_ref, out_ref, wbuf, sbuf, obuf, wsem, ssem, osem = rest
        else:
            out_ref, wbuf, sbuf, obuf, wsem, ssem, osem = rest
        nb = meta_ref[4, 0]
        pltpu.make_async_copy(w_ref.at[meta_ref[0, 0]], wbuf.at[0],
                              wsem.at[0]).start()
        pltpu.make_async_copy(src_ref.at[pl.ds(0, BLK_M)], sbuf.at[0],
                              ssem.at[0]).start()

        def step(m, carry):
            par = meta_ref[1, m]
            pb = m % 2

            @pl.when(meta_ref[2, m] == 1)
            def _():
                pltpu.make_async_copy(w_ref.at[meta_ref[0, m]], wbuf.at[par],
                                      wsem.at[par]).wait()
                nxt = meta_ref[3, m]

                @pl.when(nxt >= 0)
                def _():
                    pltpu.make_async_copy(w_ref.at[nxt], wbuf.at[1 - par],
                                          wsem.at[1 - par]).start()

            pltpu.make_async_copy(src_ref.at[pl.ds(m * BLK_M, BLK_M)],
                                  sbuf.at[pb], ssem.at[pb]).wait()

            @pl.when(m + 1 < nb)
            def _():
                pltpu.make_async_copy(
                    src_ref.at[pl.ds((m + 1) * BLK_M, BLK_M)],
                    sbuf.at[1 - pb], ssem.at[1 - pb]).start()

            @pl.when(m >= 2)
            def _():
                pltpu.make_async_copy(
                    obuf.at[pb], out_ref.at[pl.ds((m - 2) * BLK_M, BLK_M)],
                    osem.at[pb]).wait()

            eid = meta_ref[0, m]
            y = jnp.dot(sbuf[pb], wbuf[par],
                        preferred_element_type=jnp.float32) + b_ref[pl.ds(eid, 1)]
            if phase == 1:
                obuf[pb] = jnp.maximum(y, 0.0).astype(out_dtype)
            else:
                obuf[pb] = y * gs_ref[pl.ds(m, 1)][0]
            pltpu.make_async_copy(obuf.at[pb],
                                  out_ref.at[pl.ds(m * BLK_M, BLK_M)],
                                  osem.at[pb]).start()
            return carry

        lax.fori_loop(0, nb, step, 0)
        pltpu.make_async_copy(obuf.at[(nb - 1) % 2],
                              out_ref.at[pl.ds((nb - 1) * BLK_M, BLK_M)],
                              osem.at[(nb - 1) % 2]).wait()
        pltpu.make_async_copy(obuf.at[(nb - 2) % 2],
                              out_ref.at[pl.ds((nb - 2) * BLK_M, BLK_M)],
                              osem.at[(nb - 2) % 2]).wait()

    in_specs = [
        pl.BlockSpec(memory_space=pltpu.MemorySpace.SMEM),
        pl.BlockSpec(memory_space=pltpu.MemorySpace.HBM),
        pl.BlockSpec(memory_space=pltpu.MemorySpace.HBM),
        pl.BlockSpec(memory_space=pltpu.MemorySpace.VMEM),
    ]
    if phase == 2:
        in_specs.append(pl.BlockSpec(memory_space=pltpu.MemorySpace.VMEM))
    src_dtype = jnp.float32 if phase == 1 else jnp.bfloat16
    return pl.pallas_call(
        body,
        in_specs=in_specs,
        out_specs=pl.BlockSpec(memory_space=pltpu.MemorySpace.HBM),
        out_shape=jax.ShapeDtypeStruct((NSLOT, dout), out_dtype),
        scratch_shapes=[
            pltpu.VMEM((2, din, dout), jnp.float32),
            pltpu.VMEM((2, BLK_M, din), src_dtype),
            pltpu.VMEM((2, BLK_M, dout), out_dtype),
            pltpu.SemaphoreType.DMA((2,)),
            pltpu.SemaphoreType.DMA((2,)),
            pltpu.SemaphoreType.DMA((2,)),
        ],
    )


def _phase1(meta, Xg, W1, b1):
    return _make_grouped_mm(EMBED, HIDDEN, jnp.bfloat16, 1)(meta, Xg, W1, b1)


def _phase2(meta, H, W2, b2, GS):
    return _make_grouped_mm(HIDDEN, EMBED, jnp.float32, 2)(meta, H, W2, b2, GS)


@functools.lru_cache(maxsize=None)
def _make_combine():
    @functools.partial(
        pl.kernel,
        out_type=jax.ShapeDtypeStruct((S, EMBED), jnp.float32),
        mesh=_vector_mesh(),
        scratch_types=[pltpu.VMEM((2, 32), jnp.int32),
                       pltpu.VMEM((2, 32), jnp.int32),
                       pltpu.VMEM((32, EMBED), jnp.float32),
                       pltpu.VMEM((32, EMBED), jnp.float32),
                       pltpu.VMEM((32, EMBED), jnp.float32),
                       pltpu.SemaphoreType.DMA,
                       pltpu.SemaphoreType.DMA],
    )
    def _combine(yg_hbm, pos_hbm, out_hbm, p0, p1, r0, r1, ov, sem0, sem1):
        w = lax.axis_index("s") * 2 + lax.axis_index("c")
        pltpu.sync_copy(pos_hbm.at[0, w], p0)
        pltpu.sync_copy(pos_hbm.at[1, w], p1)

        @pl.loop(0, 2)
        def _(sb):
            cp0 = pltpu.async_copy(yg_hbm.at[p0.at[sb]], r0, sem0)
            cp1 = pltpu.async_copy(yg_hbm.at[p1.at[sb]], r1, sem1)
            cp0.wait()
            cp1.wait()

            @pl.loop(0, 32)
            def _(j):
                @pl.loop(0, EMBED // 16)
                def _(cc):
                    sl = pl.ds(cc * 16, 16)
                    ov[j, sl] = r0[j, sl] + r1[j, sl]

            pltpu.sync_copy(ov, out_hbm.at[pl.ds(w * 64 + sb * 32, 32)])

    return _combine


def kernel(x, Wr1, br1, Wr2, br2, W1, b1, W2, b2):
    B = x.shape[0]
    x2 = x.reshape(S, EMBED)
    Wr2p = jnp.zeros((HIDDEN, EP), jnp.float32).at[:, :NEXP].set(Wr2)
    br2c = jnp.full((EP, 1), NEG / 4, jnp.float32).at[:NEXP, 0].set(br2)

    scoresT = _router(x2, Wr1, br1.reshape(1, HIDDEN), Wr2p, br2c)
    pos, meta, GS = _routing(scoresT)
    Xg = _make_dispatch()(x2, pos.reshape(2, 16, 2, 64))
    H = _phase1(meta, Xg, W1, b1)
    Yg = _phase2(meta, H, W2, b2, GS)
    out = _make_combine()(Yg, pos.reshape(2, 32, 2, 32))
    return out.reshape(B, S, EMBED)


# speedup vs baseline: 1.4504x; 1.0080x over previous
import functools

import jax
import jax.numpy as jnp
from jax import lax
from jax.experimental import pallas as pl
from jax.experimental.pallas import tpu as pltpu
from jax.experimental.pallas import tpu_sc as plsc

EMBED = 1024
HIDDEN = 4096
NEXP = 8
S = 2048
EP = 128
BLK_M = 128
NBLK = 40
NSLOT = NBLK * BLK_M
NEG = -3.0e38

@functools.lru_cache(maxsize=None)
def _vector_mesh():
    return plsc.VectorSubcoreMesh(core_axis_name="c", subcore_axis_name="s")


def _router_routing_body(x_ref, wr1_ref, br1_ref, wr2_ref, br2_ref,
                         pos_ref, meta_ref, gs_ref, st_ref,
                         i1_ref, i2_ref, g1_ref, g2_ref, r1_ref, r2_ref):
    mb = x_ref.shape[0]
    m = pl.program_id(0)
    h = jnp.maximum(
        jnp.dot(x_ref[...], wr1_ref[...], preferred_element_type=jnp.float32)
        + br1_ref[...], 0.0)
    st = lax.dot_general(wr2_ref[...], h, (((0,), (1,)), ((), ())),
                         preferred_element_type=jnp.float32)
    st_ref[:, pl.ds(m * mb, mb)] = st + br2_ref[...]

    @pl.when(m == pl.num_programs(0) - 1)
    def _():
        _routing_compute(st_ref, pos_ref, meta_ref, gs_ref,
                         i1_ref, i2_ref, g1_ref, g2_ref, r1_ref, r2_ref)


def _routing_compute(st_ref, pos_ref, meta_ref, gs_ref,
                     i1_ref, i2_ref, g1_ref, g2_ref, r1_ref, r2_ref):
    TB = 128
    NB = S // TB
    row = lax.broadcasted_iota(jnp.int32, (EP, TB), 0)
    ut = (lax.broadcasted_iota(jnp.int32, (TB, TB), 0)
          < lax.broadcasted_iota(jnp.int32, (TB, TB), 1)).astype(jnp.float32)

    def pass1(b, carry):
        p1, p2 = carry
        sl = pl.ds(b * TB, TB)
        sb = st_ref[:, sl]
        m1 = jnp.max(sb, axis=0, keepdims=True)
        i1 = jnp.min(jnp.where(sb == m1, row, EP), axis=0, keepdims=True)
        s2 = jnp.where(row == i1, NEG, sb)
        m2 = jnp.max(s2, axis=0, keepdims=True)
        i2 = jnp.min(jnp.where(s2 == m2, row, EP), axis=0, keepdims=True)
        e = jnp.exp(m2 - m1)
        g1 = 1.0 / (1.0 + e)
        o1 = (row == i1).astype(jnp.float32)
        o2 = (row == i2).astype(jnp.float32)
        c1 = lax.dot_general(o1, ut, (((1,), (0,)), ((), ())),
                             preferred_element_type=jnp.float32) + p1
        c2 = lax.dot_general(o2, ut, (((1,), (0,)), ((), ())),
                             preferred_element_type=jnp.float32) + p2
        i1_ref[:, sl] = i1
        i2_ref[:, sl] = i2
        g1_ref[:, sl] = g1
        g2_ref[:, sl] = e * g1
        r1_ref[:, sl] = jnp.sum(o1 * c1, axis=0, keepdims=True)
        r2_ref[:, sl] = jnp.sum(o2 * c2, axis=0, keepdims=True)
        return (p1 + jnp.sum(o1, axis=1, keepdims=True),
                p2 + jnp.sum(o2, axis=1, keepdims=True))

    zero = jnp.zeros((EP, 1), jnp.float32)
    cnt1, cnt2 = lax.fori_loop(0, NB, pass1, (zero, zero))

    cnt = cnt1 + cnt2
    blocks = jnp.floor((cnt + (BLK_M - 1)) * (1.0 / BLK_M))
    lt = (lax.broadcasted_iota(jnp.int32, (EP, EP), 1)
          < lax.broadcasted_iota(jnp.int32, (EP, EP), 0)).astype(jnp.float32)
    bexc = lax.dot_general(lt, blocks, (((1,), (0,)), ((), ())),
                           preferred_element_type=jnp.float32)
    off = bexc * float(BLK_M)
    bend = (bexc + blocks).astype(jnp.int32)
    total = jnp.sum(blocks).astype(jnp.int32)

    mrow = lax.broadcasted_iota(jnp.int32, (1, NBLK), 1)
    mcl = jnp.minimum(mrow, total - 1)
    be = jnp.sum((bend <= mcl).astype(jnp.int32), axis=0, keepdims=True)
    uf = (blocks > 0.0).astype(jnp.float32)
    uexc = lax.dot_general(lt, uf, (((1,), (0,)), ((), ())),
                           preferred_element_type=jnp.float32)
    mrowf = mrow.astype(jnp.float32)
    mclf = mcl.astype(jnp.float32)
    rf = jnp.sum(uf * (bexc <= mclf).astype(jnp.float32), axis=0, keepdims=True)
    first = jnp.sum(uf * (bexc == mrowf).astype(jnp.float32), axis=0,
                    keepdims=True)
    rowidx = lax.broadcasted_iota(jnp.int32, (EP, 1), 0).astype(jnp.float32)
    eqn = uf * (uexc == rf).astype(jnp.float32)
    vsum = jnp.sum(rowidx * eqn, axis=0, keepdims=True)
    exist = jnp.sum(eqn, axis=0, keepdims=True)
    nf = jnp.where(exist > 0.0, vsum, -1.0)
    meta_ref[0:1, :] = be
    meta_ref[1:2, :] = (rf.astype(jnp.int32) - 1) % 2
    meta_ref[2:3, :] = first.astype(jnp.int32)
    meta_ref[3:4, :] = nf.astype(jnp.int32)
    meta_ref[4:5, :] = jnp.broadcast_to(total, (1, NBLK))

    def pass2(b, carry):
        sl = pl.ds(b * TB, TB)
        o1 = (row == i1_ref[:, sl]).astype(jnp.float32)
        o2 = (row == i2_ref[:, sl]).astype(jnp.float32)
        pos1 = r1_ref[:, sl] + jnp.sum(o1 * off, axis=0, keepdims=True)
        pos2 = r2_ref[:, sl] + jnp.sum(o2 * (off + cnt1), axis=0, keepdims=True)
        pos_ref[0:1, sl] = pos1.astype(jnp.int32)
        pos_ref[1:2, sl] = pos2.astype(jnp.int32)
        r1_ref[:, sl] = pos1
        r2_ref[:, sl] = pos2
        return carry

    lax.fori_loop(0, NB, pass2, 0)

    p1row = r1_ref[...]
    p2row = r2_ref[...]
    g1row = g1_ref[...]
    g2row = g2_ref[...]
    pcol = lax.broadcasted_iota(jnp.int32, (BLK_M, 1), 0).astype(jnp.float32)
    for m in range(NBLK):
        pc = pcol + float(m * BLK_M)
        eq1 = (pc == p1row).astype(jnp.float32)
        eq2 = (pc == p2row).astype(jnp.float32)
        gs = (lax.dot_general(eq1, g1row, (((1,), (1,)), ((), ())),
                              preferred_element_type=jnp.float32)
              + lax.dot_general(eq2, g2row, (((1,), (1,)), ((), ())),
                                preferred_element_type=jnp.float32))
        gs_ref[m] = gs


def _router_routing(x2, Wr1, br1r, Wr2p, br2c):
    mb = 512
    return pl.pallas_call(
        _router_routing_body,
        grid=(S // mb,),
        in_specs=[
            pl.BlockSpec((mb, EMBED), lambda m: (m, 0)),
            pl.BlockSpec((EMBED, HIDDEN), lambda m: (0, 0)),
            pl.BlockSpec((1, HIDDEN), lambda m: (0, 0)),
            pl.BlockSpec((HIDDEN, EP), lambda m: (0, 0)),
            pl.BlockSpec((EP, 1), lambda m: (0, 0)),
        ],
        out_specs=[
            pl.BlockSpec((2, S), lambda m: (0, 0)),
            pl.BlockSpec((5, NBLK), lambda m: (0, 0)),
            pl.BlockSpec((NBLK, BLK_M, 1), lambda m: (0, 0, 0)),
        ],
        out_shape=[
            jax.ShapeDtypeStruct((2, S), jnp.int32),
            jax.ShapeDtypeStruct((5, NBLK), jnp.int32),
            jax.ShapeDtypeStruct((NBLK, BLK_M, 1), jnp.float32),
        ],
        scratch_shapes=[pltpu.VMEM((EP, S), jnp.float32),
                        pltpu.VMEM((1, S), jnp.int32),
                        pltpu.VMEM((1, S), jnp.int32),
                        pltpu.VMEM((1, S), jnp.float32),
                        pltpu.VMEM((1, S), jnp.float32),
                        pltpu.VMEM((1, S), jnp.float32),
                        pltpu.VMEM((1, S), jnp.float32)],
    )(x2, Wr1, br1r, Wr2p, br2c)


@functools.lru_cache(maxsize=None)
def _make_dispatch():
    @functools.partial(
        pl.kernel,
        out_type=jax.ShapeDtypeStruct((NSLOT, EMBED), jnp.float32),
        mesh=_vector_mesh(),
        scratch_types=[pltpu.VMEM((2, 64), jnp.int32),
                       pltpu.VMEM((64, EMBED), jnp.float32),
                       pltpu.SemaphoreType.DMA],
    )
    def _dispatch(x_hbm, pos_hbm, xg_hbm, idx_v, rows_v, sem):
        w = lax.axis_index("s") * 2 + lax.axis_index("c")
        k = w // 16
        i = w % 16
        pltpu.sync_copy(pos_hbm.at[k, i], idx_v)

        @pl.loop(0, 2)
        def _(sb):
            t0 = i * 128 + sb * 64
            pltpu.sync_copy(x_hbm.at[pl.ds(t0, 64)], rows_v)
            pltpu.async_copy(rows_v, xg_hbm.at[idx_v.at[sb]], sem).wait()

    return _dispatch


def _make_grouped_mm(din, dout, out_dtype, phase):
    def body(meta_ref, src_ref, w_ref, b_ref, *rest):
        if phase == 2:
            gs_ref, out_ref, wbuf, sbuf, obuf, wsem, ssem, osem = rest
        else:
            out_ref, wbuf, sbuf, obuf, wsem, ssem, osem = rest
        nb = meta_ref[4, 0]
        pltpu.make_async_copy(w_ref.at[meta_ref[0, 0]], wbuf.at[0],
                              wsem.at[0]).start()
        pltpu.make_async_copy(src_ref.at[pl.ds(0, BLK_M)], sbuf.at[0],
                              ssem.at[0]).start()

        def step(m, carry):
            par = meta_ref[1, m]
            pb = m % 2

            @pl.when(meta_ref[2, m] == 1)
            def _():
                pltpu.make_async_copy(w_ref.at[meta_ref[0, m]], wbuf.at[par],
                                      wsem.at[par]).wait()
                nxt = meta_ref[3, m]

                @pl.when(nxt >= 0)
                def _():
                    pltpu.make_async_copy(w_ref.at[nxt], wbuf.at[1 - par],
                                          wsem.at[1 - par]).start()

            pltpu.make_async_copy(src_ref.at[pl.ds(m * BLK_M, BLK_M)],
                                  sbuf.at[pb], ssem.at[pb]).wait()

            @pl.when(m + 1 < nb)
            def _():
                pltpu.make_async_copy(
                    src_ref.at[pl.ds((m + 1) * BLK_M, BLK_M)],
                    sbuf.at[1 - pb], ssem.at[1 - pb]).start()

            @pl.when(m >= 2)
            def _():
                pltpu.make_async_copy(
                    obuf.at[pb], out_ref.at[pl.ds((m - 2) * BLK_M, BLK_M)],
                    osem.at[pb]).wait()

            eid = meta_ref[0, m]
            y = jnp.dot(sbuf[pb], wbuf[par],
                        preferred_element_type=jnp.float32) + b_ref[pl.ds(eid, 1)]
            if phase == 1:
                obuf[pb] = jnp.maximum(y, 0.0).astype(out_dtype)
            else:
                obuf[pb] = y * gs_ref[pl.ds(m, 1)][0]
            pltpu.make_async_copy(obuf.at[pb],
                                  out_ref.at[pl.ds(m * BLK_M, BLK_M)],
                                  osem.at[pb]).start()
            return carry

        lax.fori_loop(0, nb, step, 0)
        pltpu.make_async_copy(obuf.at[(nb - 1) % 2],
                              out_ref.at[pl.ds((nb - 1) * BLK_M, BLK_M)],
                              osem.at[(nb - 1) % 2]).wait()
        pltpu.make_async_copy(obuf.at[(nb - 2) % 2],
                              out_ref.at[pl.ds((nb - 2) * BLK_M, BLK_M)],
                              osem.at[(nb - 2) % 2]).wait()

    in_specs = [
        pl.BlockSpec(memory_space=pltpu.MemorySpace.SMEM),
        pl.BlockSpec(memory_space=pltpu.MemorySpace.HBM),
        pl.BlockSpec(memory_space=pltpu.MemorySpace.HBM),
        pl.BlockSpec(memory_space=pltpu.MemorySpace.VMEM),
    ]
    if phase == 2:
        in_specs.append(pl.BlockSpec(memory_space=pltpu.MemorySpace.VMEM))
    src_dtype = jnp.float32 if phase == 1 else jnp.bfloat16
    return pl.pallas_call(
        body,
        in_specs=in_specs,
        out_specs=pl.BlockSpec(memory_space=pltpu.MemorySpace.HBM),
        out_shape=jax.ShapeDtypeStruct((NSLOT, dout), out_dtype),
        scratch_shapes=[
            pltpu.VMEM((2, din, dout), jnp.float32),
            pltpu.VMEM((2, BLK_M, din), src_dtype),
            pltpu.VMEM((2, BLK_M, dout), out_dtype),
            pltpu.SemaphoreType.DMA((2,)),
            pltpu.SemaphoreType.DMA((2,)),
            pltpu.SemaphoreType.DMA((2,)),
        ],
    )


def _phase1(meta, Xg, W1, b1):
    return _make_grouped_mm(EMBED, HIDDEN, jnp.bfloat16, 1)(meta, Xg, W1, b1)


def _phase2(meta, H, W2, b2, GS):
    return _make_grouped_mm(HIDDEN, EMBED, jnp.float32, 2)(meta, H, W2, b2, GS)


@functools.lru_cache(maxsize=None)
def _make_combine():
    @functools.partial(
        pl.kernel,
        out_type=jax.ShapeDtypeStruct((S, EMBED), jnp.float32),
        mesh=_vector_mesh(),
        scratch_types=[pltpu.VMEM((2, 32), jnp.int32),
                       pltpu.VMEM((2, 32), jnp.int32),
                       pltpu.VMEM((32, EMBED), jnp.float32),
                       pltpu.VMEM((32, EMBED), jnp.float32),
                       pltpu.VMEM((32, EMBED), jnp.float32),
                       pltpu.SemaphoreType.DMA,
                       pltpu.SemaphoreType.DMA],
    )
    def _combine(yg_hbm, pos_hbm, out_hbm, p0, p1, r0, r1, ov, sem0, sem1):
        w = lax.axis_index("s") * 2 + lax.axis_index("c")
        pltpu.sync_copy(pos_hbm.at[0, w], p0)
        pltpu.sync_copy(pos_hbm.at[1, w], p1)

        @pl.loop(0, 2)
        def _(sb):
            cp0 = pltpu.async_copy(yg_hbm.at[p0.at[sb]], r0, sem0)
            cp1 = pltpu.async_copy(yg_hbm.at[p1.at[sb]], r1, sem1)
            cp0.wait()
            cp1.wait()

            @pl.loop(0, 32)
            def _(j):
                @pl.loop(0, EMBED // 16)
                def _(cc):
                    sl = pl.ds(cc * 16, 16)
                    ov[j, sl] = r0[j, sl] + r1[j, sl]

            pltpu.sync_copy(ov, out_hbm.at[pl.ds(w * 64 + sb * 32, 32)])

    return _combine


def kernel(x, Wr1, br1, Wr2, br2, W1, b1, W2, b2):
    B = x.shape[0]
    x2 = x.reshape(S, EMBED)
    Wr2p = jnp.zeros((HIDDEN, EP), jnp.float32).at[:, :NEXP].set(Wr2)
    br2c = jnp.full((EP, 1), NEG / 4, jnp.float32).at[:NEXP, 0].set(br2)

    pos, meta, GS = _router_routing(x2, Wr1, br1.reshape(1, HIDDEN), Wr2p, br2c)
    Xg = _make_dispatch()(x2, pos.reshape(2, 16, 2, 64))
    H = _phase1(meta, Xg, W1, b1)
    Yg = _phase2(meta, H, W2, b2, GS)
    out = _make_combine()(Yg, pos.reshape(2, 32, 2, 32))
    return out.reshape(B, S, EMBED)
